# Initial kernel scaffold; baseline (speedup 1.0000x reference)
import jax
import jax.numpy as jnp
from jax.experimental import pallas as pl


def kernel(node_attrs, node_feats, edge_attrs, edge_feats, edge_index, W_up, W_msg, W_skip):
    raise NotImplementedError("write your pallas kernel here")


# speedup vs baseline: 8.0609x; 8.0609x over previous
import functools

import jax
import jax.numpy as jnp
import numpy as np
from jax import lax
from jax.experimental import pallas as pl
from jax.experimental.pallas import tpu as pltpu
from jax.experimental.pallas import tpu_sc as plsc

AVG_NUM_NEIGHBORS = 16.0
_L_OF = np.repeat(np.arange(4), [1, 3, 5, 7])
_K_RANGES = [(0, 1), (1, 4), (4, 9), (9, 16)]

_NC, _NS, _LANES = 2, 16, 16
_NWORK = _NC * _NS
_W = 25
_WIN_PER_WORKER = 13
_NWIN = _NWORK * _WIN_PER_WORKER
_B = 64


def _splat(ref, flat_idx):
    return plsc.load_gather(ref, [jnp.full((_LANES,), flat_idx, jnp.int32)])


_WB_STRIDE = 32


def _sc_agg_kernel(nsph, c, h_hbm, sph_hbm, ef_hbm, send_hbm, oidx_hbm,
                   ridx_hbm, wb_hbm, agg_hbm, wb_ref, sidx_ref, oidx_ref,
                   ridx_ref, hbuf, efbuf, sphbuf, acc_ref, sem1, sem2, sem3):
    row = nsph * c
    wid = lax.axis_index("s") * _NC + lax.axis_index("c")
    pltpu.sync_copy(wb_hbm.at[pl.ds(wid * _WB_STRIDE, _WB_STRIDE)], wb_ref)

    def window_body(j, carry):
        win = wid * _WIN_PER_WORKER + j
        node_base = win * _W
        ej = wb_ref[pl.ds(j, _LANES)]
        e0 = ej[0]
        e1 = ej[1]
        e0a = (e0 // 8) * 8
        nb = jnp.where(e1 > e0, (e1 - e0a + _B - 1) // _B, 0)

        def zero_body(i, carry2):
            acc_ref[pl.ds(i * _LANES, _LANES)] = jnp.zeros(
                (_LANES,), jnp.float32)
            return carry2
        lax.fori_loop(0, _W * row // _LANES, zero_body, 0)

        def batch_body(b, carry2):
            eb = e0a + b * _B
            pltpu.sync_copy(send_hbm.at[pl.ds(eb, _B)], sidx_ref)
            pltpu.sync_copy(oidx_hbm.at[pl.ds(eb, _B)], oidx_ref)
            pltpu.sync_copy(ridx_hbm.at[pl.ds(eb, _B)], ridx_ref)
            d1 = pltpu.async_copy(h_hbm.at[sidx_ref], hbuf, sem1)
            d2 = pltpu.async_copy(ef_hbm.at[oidx_ref], efbuf, sem2)
            d3 = pltpu.async_copy(sph_hbm.at[oidx_ref], sphbuf, sem3)
            d1.wait()
            d2.wait()
            d3.wait()

            def edge_body(s, carry3):
                eidx = eb + s
                valid = jnp.logical_and(eidx >= e0, eidx < e1)
                r_b = _splat(ridx_ref, s)
                rloc = jnp.clip(r_b - node_base, 0, _W - 1)
                idx_base = rloc * row + lax.iota(jnp.int32, _LANES)
                zeros = jnp.zeros((_LANES,), jnp.float32)
                s_full = jnp.full((_LANES,), s, jnp.int32)
                sph_b = [
                    jnp.where(
                        valid,
                        plsc.load_gather(
                            sphbuf,
                            [s_full, jnp.full((_LANES,), k, jnp.int32)]),
                        zeros)
                    for k in range(nsph)]
                for cb in range(c // _LANES):
                    h_cb = hbuf[s, pl.ds(cb * _LANES, _LANES)]
                    for l, (k_lo, k_hi) in enumerate(_K_RANGES):
                        g = h_cb * efbuf[s, pl.ds(l * c + cb * _LANES,
                                                  _LANES)]
                        for k in range(k_lo, k_hi):
                            plsc.addupdate_scatter(
                                acc_ref,
                                [idx_base + (k * c + cb * _LANES)],
                                g * sph_b[k])
                return carry3
            lax.fori_loop(0, _B, edge_body, 0)
            return carry2
        lax.fori_loop(0, nb, batch_body, 0)
        pltpu.sync_copy(acc_ref, agg_hbm.at[pl.ds(node_base * row, _W * row)])
        return carry
    lax.fori_loop(0, _WIN_PER_WORKER, window_body, 0)


def _tc_up(x_ref, w_ref, o_ref):
    o_ref[...] = jnp.dot(x_ref[...], w_ref[...],
                         preferred_element_type=jnp.float32)


def _tc_out2(nl, nelem, a_ref, lm_ref, na_ref, wm_ref, ws_ref, o_ref):
    a2 = a_ref[...]
    lm = lm_ref[...]
    na = na_ref[...]
    msg = jnp.zeros(o_ref.shape, jnp.float32)
    for l in range(nl):
        msg = msg + lm[:, l:l + 1] * jnp.dot(
            a2, wm_ref[l], preferred_element_type=jnp.float32)
    acc = jnp.zeros(o_ref.shape, jnp.float32)
    for el in range(nelem):
        acc = acc + na[:, el:el + 1] * jnp.dot(
            msg, ws_ref[el], preferred_element_type=jnp.float32)
    o_ref[...] = acc


def kernel(node_attrs, node_feats, edge_attrs, edge_feats, edge_index,
           W_up, W_msg, W_skip):
    n, c = node_feats.shape
    e, nsph = edge_attrs.shape
    nelem = node_attrs.shape[1]
    nl = W_msg.shape[0]
    row = nsph * c

    sender = edge_index[0].astype(jnp.int32)
    receiver = edge_index[1].astype(jnp.int32)

    keyf = jnp.concatenate(
        [receiver.astype(jnp.float32), jnp.full((1,), 3e9, jnp.float32)])
    send_x = jnp.concatenate([sender, jnp.zeros((1,), jnp.int32)])
    iota_e = jnp.arange(e + 1, dtype=jnp.int32)
    recv_f, send_s1, order1 = lax.sort((keyf, send_x, iota_e), num_keys=1)
    recv_s = recv_f[:e].astype(jnp.int32)
    send_s = send_s1[:e]
    order = order1[:e]
    win_starts = (jnp.arange(_NWIN + 1, dtype=jnp.int32) * _W)
    bounds = jnp.searchsorted(recv_s, win_starts, side="left").astype(
        jnp.int32)
    wb_idx = (_WIN_PER_WORKER * jnp.arange(_NWORK, dtype=jnp.int32)[:, None]
              + jnp.arange(_WIN_PER_WORKER + 1, dtype=jnp.int32)[None, :])
    wb = jnp.take(bounds, wb_idx)
    wb = jnp.pad(
        wb, ((0, 0), (0, _WB_STRIDE - _WIN_PER_WORKER - 1))).reshape(-1)
    pad = _B
    send_p = jnp.pad(send_s, (0, pad))
    order_p = jnp.pad(order, (0, pad))
    recv_p = jnp.pad(recv_s, (0, pad))
    sph_p = jnp.pad(edge_attrs, ((0, 0), (0, c - nsph)))

    bn1 = 400
    h = pl.pallas_call(
        _tc_up,
        grid=(pl.cdiv(n, bn1),),
        in_specs=[pl.BlockSpec((bn1, c), lambda i: (i, 0)),
                  pl.BlockSpec((c, c), lambda i: (0, 0))],
        out_specs=pl.BlockSpec((bn1, c), lambda i: (i, 0)),
        out_shape=jax.ShapeDtypeStruct((n, c), jnp.float32),
    )(node_feats, W_up)

    mesh = plsc.VectorSubcoreMesh(core_axis_name="c", subcore_axis_name="s",
                                  num_cores=_NC, num_subcores=_NS)
    sc_fn = pl.kernel(
        functools.partial(_sc_agg_kernel, nsph, c),
        out_type=jax.ShapeDtypeStruct((_NWIN * _W * row,), jnp.float32),
        mesh=mesh,
        compiler_params=pltpu.CompilerParams(needs_layout_passes=False),
        scratch_types=[
            pltpu.VMEM((_WB_STRIDE,), jnp.int32),
            pltpu.VMEM((_B,), jnp.int32),
            pltpu.VMEM((_B,), jnp.int32),
            pltpu.VMEM((_B,), jnp.int32),
            pltpu.VMEM((_B, c), jnp.float32),
            pltpu.VMEM((_B, nl * c), jnp.float32),
            pltpu.VMEM((_B, c), jnp.float32),
            pltpu.VMEM((_W * row,), jnp.float32),
            pltpu.SemaphoreType.DMA,
            pltpu.SemaphoreType.DMA,
            pltpu.SemaphoreType.DMA,
        ],
    )
    agg_flat = sc_fn(h, sph_p, edge_feats, send_p, order_p, recv_p, wb)
    agg = agg_flat.reshape(_NWIN * _W, nsph, c)

    wm = W_msg * (1.0 / AVG_NUM_NEIGHBORS)
    agg2 = agg_flat.reshape(_NWIN * _W * nsph, c)
    lmask1 = jnp.zeros((nsph, nl), jnp.float32).at[
        jnp.arange(nsph), jnp.asarray(_L_OF)].set(1.0)
    lmask = jnp.tile(lmask1, (n, 1))
    na_rep = jnp.repeat(node_attrs, nsph, axis=0)
    br = 400 * nsph
    out2 = pl.pallas_call(
        functools.partial(_tc_out2, nl, nelem),
        grid=(n * nsph // br,),
        in_specs=[pl.BlockSpec((br, c), lambda i: (i, 0)),
                  pl.BlockSpec((br, nl), lambda i: (i, 0)),
                  pl.BlockSpec((br, nelem), lambda i: (i, 0)),
                  pl.BlockSpec((nl, c, c), lambda i: (0, 0, 0)),
                  pl.BlockSpec((nelem, c, c), lambda i: (0, 0, 0))],
        out_specs=pl.BlockSpec((br, c), lambda i: (i, 0)),
        out_shape=jax.ShapeDtypeStruct((n * nsph, c), jnp.float32),
    )(agg2, lmask, na_rep, wm, W_skip)
    return out2.reshape(n, nsph, c)
